# baseline (speedup 1.0000x reference)
import functools

import jax
import jax.numpy as jnp
from jax import lax
from jax.experimental import pallas as pl
from jax.experimental.pallas import tpu as pltpu
from jax.experimental.pallas import tpu_sc as plsc

_B = 1024
_D = 32
_R = 1000

_NC = 2
_NS = 16
_NW = _NC * _NS
_NIDX = 2 * _B
_BPW = _NIDX // _NW

_sc_mesh = plsc.VectorSubcoreMesh(core_axis_name="c", subcore_axis_name="s")


@functools.partial(
    pl.kernel,
    mesh=_sc_mesh,
    out_type=jax.ShapeDtypeStruct((_NIDX, _D), jnp.float32),
    scratch_types=[
        pltpu.VMEM((_BPW,), jnp.int32),
        pltpu.VMEM((_BPW, _D), jnp.float32),
        pltpu.SemaphoreType.DMA,
    ],
    compiler_params=pltpu.CompilerParams(use_tc_tiling_on_sc=False),
)
def _sc_gather(idx_hbm, table_hbm, out_hbm, idx_v, rows_v, sem):
    wid = lax.axis_index("s") * _NC + lax.axis_index("c")
    base = wid * _BPW
    pltpu.sync_copy(idx_hbm.at[pl.ds(base, _BPW)], idx_v)
    pltpu.async_copy(table_hbm.at[idx_v], rows_v, sem).wait()
    pltpu.sync_copy(rows_v, out_hbm.at[pl.ds(base, _BPW)])


_BB = 256


def _tc_distance_body(h_ref, t_ref, rel_t_ref, out_ref):
    d = h_ref[...] - t_ref[...]
    acc = jnp.abs(d[:, 0:1] + rel_t_ref[0:1, :])
    for k in range(1, _D):
        acc = acc + jnp.abs(d[:, k:k + 1] + rel_t_ref[k:k + 1, :])
    out_ref[...] = -acc


def _tc_distance(rows, rel_t):
    nblk = _B // _BB
    return pl.pallas_call(
        _tc_distance_body,
        grid=(nblk,),
        in_specs=[
            pl.BlockSpec((_BB, _D), lambda i: (i, 0)),
            pl.BlockSpec((_BB, _D), lambda i: (i + nblk, 0)),
            pl.BlockSpec((_D, _R), lambda i: (0, 0)),
        ],
        out_specs=pl.BlockSpec((_BB, _R), lambda i: (i, 0)),
        out_shape=jax.ShapeDtypeStruct((_B, _R), jnp.float32),
    )(rows, rows, rel_t)


def kernel(heads, tails, entity_emb, relation_emb):
    idx = jnp.concatenate([heads, tails]).astype(jnp.int32)
    rows = _sc_gather(idx, entity_emb)
    return _tc_distance(rows, relation_emb.T)
